# Initial kernel scaffold; baseline (speedup 1.0000x reference)
import jax
import jax.numpy as jnp
from jax.experimental import pallas as pl


def kernel(x):
    raise NotImplementedError("write your pallas kernel here")


# speedup vs baseline: 34.7728x; 34.7728x over previous
import functools

import jax
import jax.numpy as jnp
from jax import lax
from jax.experimental import pallas as pl
from jax.experimental.pallas import tpu as pltpu
from jax.experimental.pallas import tpu_sc as plsc

_B = 16
_H = 512
_W = 512
_OFF = 5
_IMG = _H * _W
_CHUNK = 65536
_NCHUNK = _IMG // _CHUNK
_VECS = _CHUNK // 16
_VPR = _W // 16
_PAIRS = float(_H * (_W - _OFF))


def _lane_reduce(vec, op):
    a = vec[0]
    for i in range(1, 16):
        a = op(a, vec[i])
    return a


def _glcm_body(x_hbm, out_hbm, buf, stage):
    c = lax.axis_index("c")
    s = lax.axis_index("s")
    img = c * 8 + s
    lanes = lax.iota(jnp.int32, 16)

    @pl.when(s < 8)
    def _():
        base = img * _IMG
        buf[pl.ds(_CHUNK, 16)] = jnp.zeros((16,), jnp.float32)

        def p1_chunk(k, carry):
            vmn, vmx = carry
            pltpu.sync_copy(x_hbm.at[pl.ds(base + k * _CHUNK, _CHUNK)],
                            buf.at[pl.ds(0, _CHUNK)])

            def p1_vec(j, cc):
                m0, m1 = cc
                v = buf[pl.ds(j * 16, 16)]
                return jnp.minimum(m0, v), jnp.maximum(m1, v)

            return lax.fori_loop(0, _VECS, p1_vec, (vmn, vmx))

        big = jnp.full((16,), 3.4e38, jnp.float32)
        vmn, vmx = lax.fori_loop(0, _NCHUNK, p1_chunk, (big, -big))
        mn = _lane_reduce(vmn, jnp.minimum)
        mx = _lane_reduce(vmx, jnp.maximum)
        ones = jnp.ones((16,), jnp.float32)
        scale = (ones * 255.0) / (ones * (mx - mn))
        beta = 0.5 - mn * scale

        def p2_chunk(k, acc):
            pltpu.sync_copy(x_hbm.at[pl.ds(base + k * _CHUNK, _CHUNK)],
                            buf.at[pl.ds(0, _CHUNK)])

            def p2_vec(j, a):
                v1 = buf[pl.ds(j * 16, 16)]
                v2 = buf[pl.ds(j * 16 + _OFF, 16)]
                q1 = (v1 * scale + beta).astype(jnp.int32)
                q2 = (v2 * scale + beta).astype(jnp.int32)
                d = q1 - q2
                col = (j % _VPR) * 16 + lanes
                keep = col < (_W - _OFF)
                return a + jnp.where(keep, (d * d).astype(jnp.float32), 0.0)

            return lax.fori_loop(0, _VECS, p2_vec, acc)

        acc = lax.fori_loop(0, _NCHUNK, p2_chunk, jnp.zeros((16,), jnp.float32))
        res = _lane_reduce(acc, jnp.add) * (1.0 / _PAIRS)
        stage[...] = jnp.where(lanes == 0, res, 0.0)
        pltpu.sync_copy(stage, out_hbm.at[img])


@functools.partial(jax.jit, static_argnums=())
def _glcm_sc(xflat):
    mesh = plsc.VectorSubcoreMesh(core_axis_name="c", subcore_axis_name="s")
    f = pl.kernel(
        _glcm_body,
        out_type=jax.ShapeDtypeStruct((_B, 16), jnp.float32),
        mesh=mesh,
        scratch_types=[
            pltpu.VMEM((_CHUNK + 16,), jnp.float32),
            pltpu.VMEM((16,), jnp.float32),
        ],
    )
    return f(xflat)


def kernel(x):
    xflat = x.reshape(_B * _IMG)
    rows = _glcm_sc(xflat)
    return rows[:, 0].reshape(_B, 1, 1, 1)


# speedup vs baseline: 46.6870x; 1.3426x over previous
import jax
import jax.numpy as jnp
from jax import lax
from jax.experimental import pallas as pl
from jax.experimental.pallas import tpu as pltpu
from jax.experimental.pallas import tpu_sc as plsc

_B = 16
_H = 512
_W = 512
_OFF = 5
_IMG = _H * _W
_HALF = _IMG // 2
_CHUNK = 32768
_NC1 = _IMG // _CHUNK
_NC2 = _HALF // _CHUNK
_VECS = _CHUNK // 16
_VPR = _W // 16
_PAIRS = float(_H * (_W - _OFF))


def _lane_reduce(vec, op):
    a = vec[0]
    for i in range(1, 16):
        a = op(a, vec[i])
    return a


def _glcm_body(x_hbm, out_hbm, bufa, bufb, mask, stage, sema, semb):
    c = lax.axis_index("c")
    s = lax.axis_index("s")
    img = c * 8 + lax.div(s, 2)
    half = lax.rem(s, 2)
    row = c * 16 + s
    lanes = lax.iota(jnp.int32, 16)

    img_base = img * _IMG
    half_base = img_base + half * _HALF
    bufs = [bufa, bufb]
    sems = [sema, semb]

    bufa[pl.ds(_CHUNK, 16)] = jnp.zeros((16,), jnp.float32)
    bufb[pl.ds(_CHUNK, 16)] = jnp.zeros((16,), jnp.float32)

    def mk_mask(j, _):
        col = j * 16 + lanes
        mask[pl.ds(j * 16, 16)] = jnp.where(col < (_W - _OFF),
                                            jnp.int32(-1), jnp.int32(0))
        return 0

    lax.fori_loop(0, _VPR, mk_mask, 0)

    def start(base, k):
        return pltpu.async_copy(
            x_hbm.at[pl.ds(base + k * _CHUNK, _CHUNK)],
            bufs[k % 2].at[pl.ds(0, _CHUNK)], sems[k % 2])

    cps = [start(img_base, 0), start(img_base, 1)]
    vmn = jnp.full((16,), 3.4e38, jnp.float32)
    vmx = -vmn
    for k in range(_NC1):
        cps[k % 2].wait()
        buf = bufs[k % 2]

        def p1_vec(j, cc):
            m0, m1 = cc
            v = buf[pl.ds(j * 16, 16)]
            return jnp.minimum(m0, v), jnp.maximum(m1, v)

        vmn, vmx = lax.fori_loop(0, _VECS, p1_vec, (vmn, vmx))
        if k + 2 < _NC1:
            cps[k % 2] = start(img_base, k + 2)

    mn = _lane_reduce(vmn, jnp.minimum)
    mx = _lane_reduce(vmx, jnp.maximum)
    ones = jnp.ones((16,), jnp.float32)
    scale = (ones * 255.0) / (ones * (mx - mn))
    beta = 0.5 - mn * scale

    cps = [start(half_base, 0), start(half_base, 1)]
    acc = jnp.zeros((16,), jnp.float32)
    for k in range(_NC2):
        cps[k % 2].wait()
        buf = bufs[k % 2]

        def p2_vec(j, ai):
            v1 = buf[pl.ds(j * 16, 16)]
            v2 = buf[pl.ds(j * 16 + _OFF, 16)]
            q1 = (v1 * scale + beta).astype(jnp.int32)
            q2 = (v2 * scale + beta).astype(jnp.int32)
            d = q1 - q2
            keep = mask[pl.ds(lax.rem(j, _VPR) * 16, 16)]
            return ai + (d * d & keep)

        acci = lax.fori_loop(0, _VECS, p2_vec, jnp.zeros((16,), jnp.int32))
        acc = acc + acci.astype(jnp.float32)
        if k + 2 < _NC2:
            cps[k % 2] = start(half_base, k + 2)

    part = _lane_reduce(acc, jnp.add)
    stage[...] = jnp.where(lanes == 0, part, 0.0)
    pltpu.sync_copy(stage, out_hbm.at[row])


@jax.jit
def _glcm_sc(xflat):
    mesh = plsc.VectorSubcoreMesh(core_axis_name="c", subcore_axis_name="s",
                                  num_cores=2, num_subcores=16)
    f = pl.kernel(
        _glcm_body,
        out_type=jax.ShapeDtypeStruct((32, 16), jnp.float32),
        mesh=mesh,
        scratch_types=[
            pltpu.VMEM((_CHUNK + 16,), jnp.float32),
            pltpu.VMEM((_CHUNK + 16,), jnp.float32),
            pltpu.VMEM((_W,), jnp.int32),
            pltpu.VMEM((16,), jnp.float32),
            pltpu.SemaphoreType.DMA,
            pltpu.SemaphoreType.DMA,
        ],
    )
    return f(xflat)


def kernel(x):
    xflat = x.reshape(_B * _IMG)
    rows = _glcm_sc(xflat)
    parts = rows[:, 0].reshape(2, 8, 2)
    sums = parts.sum(axis=2).reshape(_B)
    return (sums * (1.0 / _PAIRS)).reshape(_B, 1, 1, 1)


# speedup vs baseline: 60.4063x; 1.2939x over previous
import jax
import jax.numpy as jnp
from jax import lax
from jax.experimental import pallas as pl
from jax.experimental.pallas import tpu as pltpu
from jax.experimental.pallas import tpu_sc as plsc

_B = 16
_H = 512
_W = 512
_OFF = 5
_IMG = _H * _W
_HALF = _IMG // 2
_CHUNK = 32768
_NC1 = _IMG // _CHUNK
_NC2 = _HALF // _CHUNK
_VECS = _CHUNK // 16
_VPR = _W // 16
_PAIRS = float(_H * (_W - _OFF))


def _lane_reduce(vec, op):
    a = vec[0]
    for i in range(1, 16):
        a = op(a, vec[i])
    return a


def _glcm_body(x_hbm, out_hbm, bufa, bufb, mask, stage, mm, mm_peer, sh_mm,
               sema, semb):
    c = lax.axis_index("c")
    s = lax.axis_index("s")
    img = c * 8 + lax.div(s, 2)
    half = lax.rem(s, 2)
    row = c * 16 + s
    lanes = lax.iota(jnp.int32, 16)

    img_base = img * _IMG
    half_base = img_base + half * _HALF
    bufs = [bufa, bufb]
    sems = [sema, semb]

    bufa[pl.ds(_CHUNK, 16)] = jnp.zeros((16,), jnp.float32)
    bufb[pl.ds(_CHUNK, 16)] = jnp.zeros((16,), jnp.float32)

    def mk_mask(j, _):
        col = j * 16 + lanes
        mask[pl.ds(j * 16, 16)] = jnp.where(col < (_W - _OFF),
                                            jnp.int32(-1), jnp.int32(0))
        return 0

    lax.fori_loop(0, _VPR, mk_mask, 0)

    def start(base, k):
        return pltpu.async_copy(
            x_hbm.at[pl.ds(base + k * _CHUNK, _CHUNK)],
            bufs[k % 2].at[pl.ds(0, _CHUNK)], sems[k % 2])

    cps = [start(half_base, 0), start(half_base, 1)]
    vmn = jnp.full((16,), 3.4e38, jnp.float32)
    vmx = -vmn
    for k in range(_NC2):
        cps[k % 2].wait()
        buf = bufs[k % 2]

        def p1_vec(j, cc):
            m0, m1 = cc
            v = buf[pl.ds(j * 16, 16)]
            return jnp.minimum(m0, v), jnp.maximum(m1, v)

        vmn, vmx = lax.fori_loop(0, _VECS, p1_vec, (vmn, vmx))
        if k + 2 < _NC2:
            cps[k % 2] = start(half_base, k + 2)

    mm[0, :] = vmn
    mm[1, :] = vmx
    pltpu.sync_copy(mm, sh_mm.at[s])
    plsc.subcore_barrier()
    pltpu.sync_copy(sh_mm.at[s ^ 1], mm_peer)
    gmn = jnp.minimum(vmn, mm_peer[0, :])
    gmx = jnp.maximum(vmx, mm_peer[1, :])
    mn = _lane_reduce(gmn, jnp.minimum)
    mx = _lane_reduce(gmx, jnp.maximum)
    ones = jnp.ones((16,), jnp.float32)
    scale = (ones * 255.0) / (ones * (mx - mn))
    beta = 0.5 - mn * scale

    cps = [start(half_base, 0), start(half_base, 1)]
    acc = jnp.zeros((16,), jnp.float32)
    for k in range(_NC2):
        cps[k % 2].wait()
        buf = bufs[k % 2]

        def p2_vec(j, ai):
            v1 = buf[pl.ds(j * 16, 16)]
            v2 = buf[pl.ds(j * 16 + _OFF, 16)]
            q1 = (v1 * scale + beta).astype(jnp.int32)
            q2 = (v2 * scale + beta).astype(jnp.int32)
            d = q1 - q2
            keep = mask[pl.ds(lax.rem(j, _VPR) * 16, 16)]
            return ai + (d * d & keep)

        acci = lax.fori_loop(0, _VECS, p2_vec, jnp.zeros((16,), jnp.int32))
        acc = acc + acci.astype(jnp.float32)
        if k + 2 < _NC2:
            cps[k % 2] = start(half_base, k + 2)

    part = _lane_reduce(acc, jnp.add)
    stage[...] = jnp.where(lanes == 0, part, 0.0)
    pltpu.sync_copy(stage, out_hbm.at[row])


@jax.jit
def _glcm_sc(xflat):
    mesh = plsc.VectorSubcoreMesh(core_axis_name="c", subcore_axis_name="s",
                                  num_cores=2, num_subcores=16)
    f = pl.kernel(
        _glcm_body,
        out_type=jax.ShapeDtypeStruct((32, 16), jnp.float32),
        mesh=mesh,
        scratch_types=[
            pltpu.VMEM((_CHUNK + 16,), jnp.float32),
            pltpu.VMEM((_CHUNK + 16,), jnp.float32),
            pltpu.VMEM((_W,), jnp.int32),
            pltpu.VMEM((16,), jnp.float32),
            pltpu.VMEM((8, 16), jnp.float32),
            pltpu.VMEM((8, 16), jnp.float32),
            pltpu.VMEM_SHARED((16, 8, 16), jnp.float32),
            pltpu.SemaphoreType.DMA,
            pltpu.SemaphoreType.DMA,
        ],
    )
    return f(xflat)


def kernel(x):
    xflat = x.reshape(_B * _IMG)
    rows = _glcm_sc(xflat)
    parts = rows[:, 0].reshape(2, 8, 2)
    sums = parts.sum(axis=2).reshape(_B)
    return (sums * (1.0 / _PAIRS)).reshape(_B, 1, 1, 1)


# speedup vs baseline: 89.6739x; 1.4845x over previous
import jax
import jax.numpy as jnp
from jax import lax
from jax.experimental import pallas as pl
from jax.experimental.pallas import tpu as pltpu
from jax.experimental.pallas import tpu_sc as plsc

_B = 16
_H = 512
_W = 512
_OFF = 5
_IMG = _H * _W
_HALF = _IMG // 2
_CHUNK = 32768
_NC1 = _IMG // _CHUNK
_NC2 = _HALF // _CHUNK
_VECS = _CHUNK // 16
_VPR = _W // 16
_PAIRS = float(_H * (_W - _OFF))


def _lane_reduce(vec, op):
    a = vec[0]
    for i in range(1, 16):
        a = op(a, vec[i])
    return a


def _glcm_body(x_hbm, out_hbm, bufa, bufb, mask, stage, mm, mm_peer, sh_mm,
               sema, semb):
    c = lax.axis_index("c")
    s = lax.axis_index("s")
    img = c * 8 + lax.div(s, 2)
    half = lax.rem(s, 2)
    row = c * 16 + s
    lanes = lax.iota(jnp.int32, 16)

    img_base = img * _IMG
    half_base = img_base + half * _HALF
    bufs = [bufa, bufb]
    sems = [sema, semb]

    bufa[pl.ds(_CHUNK, 16)] = jnp.zeros((16,), jnp.float32)
    bufb[pl.ds(_CHUNK, 16)] = jnp.zeros((16,), jnp.float32)

    def mk_mask(j, _):
        col = j * 16 + lanes
        mask[pl.ds(j * 16, 16)] = jnp.where(col < (_W - _OFF),
                                            jnp.int32(-1), jnp.int32(0))
        return 0

    lax.fori_loop(0, _VPR, mk_mask, 0)

    def start(base, k):
        return pltpu.async_copy(
            x_hbm.at[pl.ds(base + k * _CHUNK, _CHUNK)],
            bufs[k % 2].at[pl.ds(0, _CHUNK)], sems[k % 2])

    cps = [start(half_base, 0), start(half_base, 1)]
    vmn = jnp.full((16,), 3.4e38, jnp.float32)
    vmx = -vmn
    for k in range(_NC2):
        cps[k % 2].wait()
        buf = bufs[k % 2]

        def p1_vec(j, cc):
            m0, m1 = cc
            v = buf[pl.ds(j * 16, 16)]
            return jnp.minimum(m0, v), jnp.maximum(m1, v)

        vmn, vmx = lax.fori_loop(0, _VECS, p1_vec, (vmn, vmx), unroll=4)
        if k + 2 < _NC2:
            cps[k % 2] = start(half_base, k + 2)

    mm[0, :] = vmn
    mm[1, :] = vmx
    pltpu.sync_copy(mm, sh_mm.at[s])
    plsc.subcore_barrier()
    pltpu.sync_copy(sh_mm.at[s ^ 1], mm_peer)
    gmn = jnp.minimum(vmn, mm_peer[0, :])
    gmx = jnp.maximum(vmx, mm_peer[1, :])
    mn = _lane_reduce(gmn, jnp.minimum)
    mx = _lane_reduce(gmx, jnp.maximum)
    ones = jnp.ones((16,), jnp.float32)
    scale = (ones * 255.0) / (ones * (mx - mn))
    beta = 0.5 - mn * scale

    cps = [start(half_base, 0), start(half_base, 1)]
    acc = jnp.zeros((16,), jnp.float32)
    for k in range(_NC2):
        cps[k % 2].wait()
        buf = bufs[k % 2]

        def p2_vec(j, ai):
            v1 = buf[pl.ds(j * 16, 16)]
            v2 = buf[pl.ds(j * 16 + _OFF, 16)]
            q1 = (v1 * scale + beta).astype(jnp.int32)
            q2 = (v2 * scale + beta).astype(jnp.int32)
            d = q1 - q2
            keep = mask[pl.ds(lax.rem(j, _VPR) * 16, 16)]
            return ai + (d * d & keep)

        acci = lax.fori_loop(0, _VECS, p2_vec, jnp.zeros((16,), jnp.int32), unroll=4)
        acc = acc + acci.astype(jnp.float32)
        if k + 2 < _NC2:
            cps[k % 2] = start(half_base, k + 2)

    part = _lane_reduce(acc, jnp.add)
    stage[...] = jnp.where(lanes == 0, part, 0.0)
    pltpu.sync_copy(stage, out_hbm.at[row])


@jax.jit
def _glcm_sc(xflat):
    mesh = plsc.VectorSubcoreMesh(core_axis_name="c", subcore_axis_name="s",
                                  num_cores=2, num_subcores=16)
    f = pl.kernel(
        _glcm_body,
        out_type=jax.ShapeDtypeStruct((32, 16), jnp.float32),
        mesh=mesh,
        scratch_types=[
            pltpu.VMEM((_CHUNK + 16,), jnp.float32),
            pltpu.VMEM((_CHUNK + 16,), jnp.float32),
            pltpu.VMEM((_W,), jnp.int32),
            pltpu.VMEM((16,), jnp.float32),
            pltpu.VMEM((8, 16), jnp.float32),
            pltpu.VMEM((8, 16), jnp.float32),
            pltpu.VMEM_SHARED((16, 8, 16), jnp.float32),
            pltpu.SemaphoreType.DMA,
            pltpu.SemaphoreType.DMA,
        ],
    )
    return f(xflat)


def kernel(x):
    xflat = x.reshape(_B * _IMG)
    rows = _glcm_sc(xflat)
    parts = rows[:, 0].reshape(2, 8, 2)
    sums = parts.sum(axis=2).reshape(_B)
    return (sums * (1.0 / _PAIRS)).reshape(_B, 1, 1, 1)
